# baseline (speedup 1.0000x reference)
import functools

import jax
import jax.numpy as jnp
from jax import lax
from jax.experimental import pallas as pl
from jax.experimental.pallas import tpu as pltpu
from jax.experimental.pallas import tpu_sc as plsc

_LANES = 16


def _sc_mask_body(w, s_pad, sites_hbm, mask_hbm, sites_v, mask_v):
    wid = lax.axis_index("s") * 2 + lax.axis_index("c")

    @pl.when(wid == 0)
    def _():
        pltpu.sync_copy(sites_hbm, sites_v)
        for i in range(w // _LANES):
            mask_v[pl.ds(i * _LANES, _LANES)] = jnp.ones((_LANES,), jnp.float32)
        for i in range(s_pad // _LANES):
            idx = sites_v[pl.ds(i * _LANES, _LANES)]
            plsc.store_scatter(mask_v, [idx], jnp.zeros((_LANES,), jnp.float32))
        pltpu.sync_copy(mask_v, mask_hbm)


def _tc_mul_body(mask_ref, x_ref, o_ref):
    o_ref[...] = x_ref[...] * mask_ref[...][None]


def kernel(x, mask_sites):
    c, h, w = x.shape
    n = mask_sites.shape[0]
    s_pad = (n + _LANES - 1) // _LANES * _LANES
    sites = mask_sites.astype(jnp.int32)
    sites = jnp.concatenate([sites, jnp.broadcast_to(sites[:1], (s_pad - n,))])

    sc_mask = functools.partial(
        pl.kernel,
        mesh=plsc.VectorSubcoreMesh(core_axis_name="c", subcore_axis_name="s"),
        compiler_params=pltpu.CompilerParams(needs_layout_passes=False),
        out_type=jax.ShapeDtypeStruct((w,), jnp.float32),
        scratch_types=[
            pltpu.VMEM((s_pad,), jnp.int32),
            pltpu.VMEM((w,), jnp.float32),
        ],
    )(functools.partial(_sc_mask_body, w, s_pad))
    mask = sc_mask(sites).reshape(1, w)

    bc = 12
    return pl.pallas_call(
        _tc_mul_body,
        grid=(c // bc,),
        in_specs=[
            pl.BlockSpec((1, w), lambda i: (0, 0)),
            pl.BlockSpec((bc, h, w), lambda i: (i, 0, 0)),
        ],
        out_specs=pl.BlockSpec((bc, h, w), lambda i: (i, 0, 0)),
        out_shape=jax.ShapeDtypeStruct((c, h, w), x.dtype),
    )(mask, x)


# speedup vs baseline: 1.0090x; 1.0090x over previous
import functools

import jax
import jax.numpy as jnp
from jax import lax
from jax.experimental import pallas as pl
from jax.experimental.pallas import tpu as pltpu
from jax.experimental.pallas import tpu_sc as plsc

_LANES = 16


def _sc_mask_body(w, s_pad, sites_hbm, mask_hbm, sites_v, mask_v):
    wid = lax.axis_index("s") * 2 + lax.axis_index("c")

    @pl.when(wid == 0)
    def _():
        pltpu.sync_copy(sites_hbm, sites_v)
        for i in range(w // _LANES):
            mask_v[pl.ds(i * _LANES, _LANES)] = jnp.ones((_LANES,), jnp.float32)
        for i in range(s_pad // _LANES):
            idx = sites_v[pl.ds(i * _LANES, _LANES)]
            plsc.store_scatter(mask_v, [idx], jnp.zeros((_LANES,), jnp.float32))
        pltpu.sync_copy(mask_v, mask_hbm.at[0])


def _tc_mul_body(mask_ref, x_ref, o_ref):
    o_ref[...] = x_ref[...] * mask_ref[...][None]


def kernel(x, mask_sites):
    c, h, w = x.shape
    n = mask_sites.shape[0]
    s_pad = (n + _LANES - 1) // _LANES * _LANES
    sites = mask_sites.astype(jnp.int32)
    sites = jnp.concatenate([sites, jnp.broadcast_to(sites[:1], (s_pad - n,))])

    sc_mask = functools.partial(
        pl.kernel,
        mesh=plsc.VectorSubcoreMesh(
            core_axis_name="c", subcore_axis_name="s", num_cores=1
        ),
        compiler_params=pltpu.CompilerParams(needs_layout_passes=False),
        out_type=jax.ShapeDtypeStruct((1, w), jnp.float32),
        scratch_types=[
            pltpu.VMEM((s_pad,), jnp.int32),
            pltpu.VMEM((w,), jnp.float32),
        ],
    )(functools.partial(_sc_mask_body, w, s_pad))
    mask = sc_mask(sites)

    bc = 12
    return pl.pallas_call(
        _tc_mul_body,
        grid=(c // bc,),
        in_specs=[
            pl.BlockSpec((1, w), lambda i: (0, 0)),
            pl.BlockSpec((bc, h, w), lambda i: (i, 0, 0)),
        ],
        out_specs=pl.BlockSpec((bc, h, w), lambda i: (i, 0, 0)),
        out_shape=jax.ShapeDtypeStruct((c, h, w), x.dtype),
    )(mask, x)
